# speedup vs baseline: 1.0567x; 1.0567x over previous
import functools

import jax
import jax.numpy as jnp
from jax.experimental import pallas as pl
from jax.experimental.pallas import tpu as pltpu

N = 10000
E = 320000


def _mm_proj_body(x_ref, w_ref, al_ref, ar_ref, feat_ref, el_ref, er_ref, *, heads, odim):
    x = x_ref[...]
    w = w_ref[...]
    feat = jnp.dot(x, w, preferred_element_type=jnp.float32)
    feat_ref[...] = feat
    bm = feat.shape[0]
    f3 = feat.reshape(bm, heads, odim)
    el_ref[...] = jnp.sum(f3 * al_ref[...][None], axis=-1)
    er_ref[...] = jnp.sum(f3 * ar_ref[...][None], axis=-1)


def _mm_proj(x, w, al, ar, heads, odim, bm=1000):
    n, k = x.shape
    m = w.shape[1]
    grid = (n // bm,)
    kern = functools.partial(_mm_proj_body, heads=heads, odim=odim)
    return pl.pallas_call(
        kern,
        grid=grid,
        in_specs=[
            pl.BlockSpec((bm, k), lambda i: (i, 0)),
            pl.BlockSpec((k, m), lambda i: (0, 0)),
            pl.BlockSpec((heads, odim), lambda i: (0, 0)),
            pl.BlockSpec((heads, odim), lambda i: (0, 0)),
        ],
        out_specs=[
            pl.BlockSpec((bm, m), lambda i: (i, 0)),
            pl.BlockSpec((bm, heads), lambda i: (i, 0)),
            pl.BlockSpec((bm, heads), lambda i: (i, 0)),
        ],
        out_shape=[
            jax.ShapeDtypeStruct((n, m), jnp.float32),
            jax.ShapeDtypeStruct((n, heads), jnp.float32),
            jax.ShapeDtypeStruct((n, heads), jnp.float32),
        ],
    )(x, w, al, ar)


def _layer(x, src, dst, W, al, ar, b, heads, odim, act):
    n = x.shape[0]
    feat, el, er = _mm_proj(x, W, al, ar, heads, odim)
    feat = feat.reshape(n, heads, odim)
    elmax = jnp.max(el, axis=0)
    mhat = jax.nn.leaky_relu(elmax[None, :] + er, negative_slope=0.2)
    e = jax.nn.leaky_relu(el[src] + er[dst], negative_slope=0.2)
    ex = jnp.exp(e - mhat[dst])
    den = jax.ops.segment_sum(ex, dst, num_segments=n)
    msg = feat[src] * ex[:, :, None]
    rst = jax.ops.segment_sum(msg, dst, num_segments=n)
    rst = rst / (den[:, :, None] + 1e-9)
    rst = rst + b.reshape(1, heads, odim)
    if act:
        rst = jax.nn.elu(rst)
    return rst


def kernel(inputs, edge_index, W0, al0, ar0, b0, W1, al1, ar1, b1, W2, al2, ar2, b2, W3, al3, ar3, b3):
    src = edge_index[0]
    dst = edge_index[1]
    h = _layer(inputs, src, dst, W0, al0, ar0, b0, 8, 128, True).reshape(N, -1)
    h = _layer(h, src, dst, W1, al1, ar1, b1, 8, 128, True).reshape(N, -1)
    h = _layer(h, src, dst, W2, al2, ar2, b2, 8, 128, True).reshape(N, -1)
    h = _layer(h, src, dst, W3, al3, ar3, b3, 1, 128, False).mean(axis=1)
    return h


# speedup vs baseline: 6.7019x; 6.3425x over previous
import functools

import jax
import jax.numpy as jnp
from jax import lax
from jax.experimental import pallas as pl
from jax.experimental.pallas import tpu as pltpu
from jax.experimental.pallas import tpu_sc as plsc

N = 10000
E = 320000
NPAD = 10240
W = 128
NW = 32
EPW = 10112
EPAD = NW * EPW
B = 32
NB = EPW // B
NBUF = 2
ROWS_PER_TILE = NPAD // 16


def _mm_body(x_ref, w_ref, al_ref, ar_ref, feat_ref, el_ref, er_ref, *, heads):
    x = x_ref[...]
    w = w_ref[...]
    feat = jnp.dot(x, w, preferred_element_type=jnp.float32)
    bm = feat.shape[0]
    f3 = feat.reshape(bm, heads, 128)
    el_ref[...] = jnp.sum(f3 * al_ref[...][None], axis=-1)
    er_ref[...] = jnp.sum(f3 * ar_ref[...][None], axis=-1)
    for h in range(heads):
        feat_ref[h, :, :] = feat[:, h * 128:(h + 1) * 128]


def _mm_proj(x, w, al, ar, heads, bm=1000):
    n, k = x.shape
    m = w.shape[1]
    kern = functools.partial(_mm_body, heads=heads)
    return pl.pallas_call(
        kern,
        grid=(n // bm,),
        in_specs=[
            pl.BlockSpec((bm, k), lambda i: (i, 0)),
            pl.BlockSpec((k, m), lambda i: (0, 0)),
            pl.BlockSpec((heads, 128), lambda i: (0, 0)),
            pl.BlockSpec((heads, 128), lambda i: (0, 0)),
        ],
        out_specs=[
            pl.BlockSpec((heads, bm, W), lambda i: (0, i, 0)),
            pl.BlockSpec((bm, heads), lambda i: (i, 0)),
            pl.BlockSpec((bm, heads), lambda i: (i, 0)),
        ],
        out_shape=[
            jax.ShapeDtypeStruct((heads, NPAD, W), jnp.float32),
            jax.ShapeDtypeStruct((n, heads), jnp.float32),
            jax.ShapeDtypeStruct((n, heads), jnp.float32),
        ],
    )(x, w, al, ar)


def _edge_body(heads, feat2, elt, ert, emxt, src_h, dst_h, out2, outd,
               acc, el_v, er_v, emx_v, den_v, srcb, dstb, gidx, rows, dlb, wb,
               gs0, ss0):
    c = lax.axis_index("c")
    s = lax.axis_index("s")
    wid = s * 2 + c
    ebase = wid * EPW

    def head_body(h, _):
        def _z(i, _):
            for cc in range(W // 16):
                rows[0, i, pl.ds(cc * 16, 16)] = jnp.zeros((16,), jnp.float32)
            return 0
        lax.fori_loop(0, B, _z, 0)

        def _zd(i, _):
            den_v[pl.ds(i * 16, 16)] = jnp.zeros((16,), jnp.float32)
            return 0
        lax.fori_loop(0, NPAD // 16, _zd, 0)

        for j in range(ROWS_PER_TILE // B):
            sl = pl.ds(s * ROWS_PER_TILE + j * B, B)
            pltpu.sync_copy(rows.at[0], acc.at[sl])

        pltpu.sync_copy(elt.at[h], el_v)
        pltpu.sync_copy(ert.at[h], er_v)
        pltpu.sync_copy(emxt.at[h], emx_v)
        hoff = h * NPAD

        plsc.subcore_barrier()

        def batch_body(b, _):
            esl = pl.ds(ebase + b * B, B)
            pltpu.sync_copy(src_h.at[esl], srcb)
            pltpu.sync_copy(dst_h.at[esl], dstb)
            for k in range(B // 16):
                sl = pl.ds(k * 16, 16)
                gidx[sl] = srcb[sl] + hoff
            pltpu.async_copy(feat2.at[gidx], rows.at[0], gs0).wait()

            emx = emx_v[:]
            for k in range(B // 16):
                sl = pl.ds(k * 16, 16)
                sv = srcb[sl]
                dv = dstb[sl]
                elv = plsc.load_gather(el_v, [sv])
                erv = plsc.load_gather(er_v, [dv])
                z = elv + erv
                e = jnp.maximum(z, 0.2 * z)
                z2 = emx + erv
                mhv = jnp.maximum(z2, 0.2 * z2)
                wv = jnp.exp(e - mhv)
                dlb[sl] = dv
                plsc.addupdate_scatter(den_v, [dv], wv)
                for j in range(16):
                    wb[k * 16 + j, :] = jnp.full((16,), wv[j], jnp.float32)

            def _mul(i, _):
                w_s = wb[i, :]
                for cc in range(W // 16):
                    sl2 = pl.ds(cc * 16, 16)
                    rows[0, i, sl2] = rows[0, i, sl2] * w_s
                return 0
            lax.fori_loop(0, B, _mul, 0)

            pltpu.async_copy(rows.at[0], acc.at[dlb], ss0, add=True).wait()
            return 0
        lax.fori_loop(0, NB, batch_body, 0)

        plsc.subcore_barrier()

        row0 = (c * heads + h) * NPAD + s * ROWS_PER_TILE
        pltpu.sync_copy(acc.at[pl.ds(s * ROWS_PER_TILE, ROWS_PER_TILE)],
                        out2.at[pl.ds(row0, ROWS_PER_TILE)])
        pltpu.sync_copy(den_v, outd.at[pl.ds((wid * heads + h) * NPAD, NPAD)])
        plsc.subcore_barrier()
        return 0

    lax.fori_loop(0, heads, head_body, 0)


def _edge_call(heads):
    mesh = plsc.VectorSubcoreMesh(core_axis_name="c", subcore_axis_name="s")
    return pl.kernel(
        functools.partial(_edge_body, heads),
        out_type=(jax.ShapeDtypeStruct((2 * heads * NPAD, W), jnp.float32),
                  jax.ShapeDtypeStruct((NW * heads * NPAD,), jnp.float32)),
        mesh=mesh,
        compiler_params=pltpu.CompilerParams(needs_layout_passes=False),
        scratch_types=[
            pltpu.VMEM_SHARED((NPAD, W), jnp.float32),
            pltpu.VMEM((NPAD,), jnp.float32),
            pltpu.VMEM((NPAD,), jnp.float32),
            pltpu.VMEM((16,), jnp.float32),
            pltpu.VMEM((NPAD,), jnp.float32),
            pltpu.VMEM((B,), jnp.int32),
            pltpu.VMEM((B,), jnp.int32),
            pltpu.VMEM((B,), jnp.int32),
            pltpu.VMEM((1, B, W), jnp.float32),
            pltpu.VMEM((B,), jnp.int32),
            pltpu.VMEM((B, 16), jnp.float32),
            pltpu.SemaphoreType.DMA,
            pltpu.SemaphoreType.DMA,
        ],
    )


_EDGE_K = {8: _edge_call(8), 1: _edge_call(1)}


def _layer(x, srcp, dstp, Wm, al, ar, b, heads, act):
    feat3, el, er = _mm_proj(x, Wm, al, ar, heads)
    elmax = jnp.max(el, axis=0)
    emxt = jnp.broadcast_to(elmax[:, None], (heads, 16))
    pad = ((0, 0), (0, NPAD - N))
    elt = jnp.pad(el.T, pad)
    ert = jnp.pad(er.T, pad)
    out2, outd = _EDGE_K[heads](feat3.reshape(heads * NPAD, W), elt, ert, emxt,
                                srcp, dstp)
    o = out2.reshape(2, heads, NPAD, W).sum(axis=0)[:, :N, :]
    den = outd.reshape(NW, heads, NPAD)[:, :, :N].sum(axis=0)
    rst = o / (den[:, :, None] + 1e-9)
    rst = rst + b.reshape(heads, 1, 128)
    if act:
        rst = jax.nn.elu(rst)
    return rst


def kernel(inputs, edge_index, W0, al0, ar0, b0, W1, al1, ar1, b1, W2, al2, ar2, b2, W3, al3, ar3, b3):
    src = edge_index[0]
    dst = edge_index[1]
    srcp = jnp.concatenate([src, jnp.zeros((EPAD - E,), jnp.int32)])
    dstp = jnp.concatenate([dst, jnp.full((EPAD - E,), NPAD - 1, jnp.int32)])

    def nxt(rst):
        return rst.transpose(1, 0, 2).reshape(N, -1)

    h = nxt(_layer(inputs, srcp, dstp, W0, al0, ar0, b0, 8, True))
    h = nxt(_layer(h, srcp, dstp, W1, al1, ar1, b1, 8, True))
    h = nxt(_layer(h, srcp, dstp, W2, al2, ar2, b2, 8, True))
    h = _layer(h, srcp, dstp, W3, al3, ar3, b3, 1, False)[0]
    return h


# speedup vs baseline: 10.9400x; 1.6324x over previous
import functools

import jax
import jax.numpy as jnp
from jax import lax
from jax.experimental import pallas as pl
from jax.experimental.pallas import tpu as pltpu
from jax.experimental.pallas import tpu_sc as plsc

N = 10000
E = 320000
NPAD = 10240
W = 128
NW = 32
EPW = 10240
EPAD = NW * EPW
B = 32
NB = EPW // B
NBUF = 2
ROWS_PER_TILE = NPAD // 16


def _mm_body(x_ref, w_ref, al_ref, ar_ref, feat_ref, el_ref, er_ref, *, heads):
    x = x_ref[...]
    w = w_ref[...]
    feat = jnp.dot(x, w, preferred_element_type=jnp.float32)
    bm = feat.shape[0]
    f3 = feat.reshape(bm, heads, 128)
    el_ref[...] = jnp.sum(f3 * al_ref[...][None], axis=-1)
    er_ref[...] = jnp.sum(f3 * ar_ref[...][None], axis=-1)
    for h in range(heads):
        feat_ref[h, :, :] = feat[:, h * 128:(h + 1) * 128]


def _mm_proj(x, w, al, ar, heads, bm=1000):
    n, k = x.shape
    m = w.shape[1]
    kern = functools.partial(_mm_body, heads=heads)
    return pl.pallas_call(
        kern,
        grid=(n // bm,),
        in_specs=[
            pl.BlockSpec((bm, k), lambda i: (i, 0)),
            pl.BlockSpec((k, m), lambda i: (0, 0)),
            pl.BlockSpec((heads, 128), lambda i: (0, 0)),
            pl.BlockSpec((heads, 128), lambda i: (0, 0)),
        ],
        out_specs=[
            pl.BlockSpec((heads, bm, W), lambda i: (0, i, 0)),
            pl.BlockSpec((bm, heads), lambda i: (i, 0)),
            pl.BlockSpec((bm, heads), lambda i: (i, 0)),
        ],
        out_shape=[
            jax.ShapeDtypeStruct((heads, NPAD, W), jnp.float32),
            jax.ShapeDtypeStruct((n, heads), jnp.float32),
            jax.ShapeDtypeStruct((n, heads), jnp.float32),
        ],
    )(x, w, al, ar)


CHB = 4
NCH = NB // CHB


def _edge_body(heads, feat2, elt, ert, emxt, src_h, dst_h, out2, outd,
               acc, el_v, er_v, emx_v, den_v, srcst, dstst, gidx, rows, dlb,
               wb, gs0, gs1, ss0, ss1, st0, st1):
    gsems = (gs0, gs1)
    ssems = (ss0, ss1)
    stsems = (st0, st1)
    c = lax.axis_index("c")
    s = lax.axis_index("s")
    wid = s * 2 + c
    ebase = wid * EPW

    def stage_start(ch, buf):
        sl = pl.ds(ebase + ch * CHB * B, CHB * B)
        pltpu.async_copy(src_h.at[sl], srcst.at[buf], stsems[buf])
        pltpu.async_copy(dst_h.at[sl], dstst.at[buf], stsems[buf])

    def stage_wait(ch, buf):
        sl = pl.ds(ebase + ch * CHB * B, CHB * B)
        pltpu.make_async_copy(src_h.at[sl], srcst.at[buf], stsems[buf]).wait()
        pltpu.make_async_copy(dst_h.at[sl], dstst.at[buf], stsems[buf]).wait()

    def gidx_fill(cbuf, bi, buf, hoff):
        for k in range(B // 16):
            sl = pl.ds(k * 16, 16)
            gidx[buf, sl] = srcst[cbuf, pl.ds(bi * B + k * 16, 16)] + hoff

    def gather_start(buf):
        pltpu.async_copy(feat2.at[gidx.at[buf]], rows.at[buf], gsems[buf])

    def gather_wait(buf):
        pltpu.make_async_copy(
            feat2.at[gidx.at[buf]], rows.at[buf], gsems[buf]).wait()

    def scatter_start(buf):
        pltpu.async_copy(
            rows.at[buf], acc.at[dlb.at[buf]], ssems[buf], add=True)

    def scatter_wait(buf):
        pltpu.make_async_copy(
            rows.at[buf], acc.at[dlb.at[buf]], ssems[buf]).wait()

    def compute(cbuf, bi, buf, hoff):
        emx = emx_v[:]
        for k in range(B // 16):
            sl = pl.ds(k * 16, 16)
            esl = pl.ds(bi * B + k * 16, 16)
            sv = srcst[cbuf, esl]
            dv = dstst[cbuf, esl]
            elv = plsc.load_gather(el_v, [sv])
            erv = plsc.load_gather(er_v, [dv])
            z = elv + erv
            e = jnp.maximum(z, 0.2 * z)
            z2 = emx + erv
            mhv = jnp.maximum(z2, 0.2 * z2)
            wv = jnp.exp(e - mhv)
            dlb[buf, sl] = dv
            plsc.addupdate_scatter(den_v, [dv], wv)
            for j in range(16):
                wb[buf, k * 16 + j, :] = jnp.full((16,), wv[j], jnp.float32)

        def _mul(i, _):
            w_s = wb[buf, i, :]
            for cc in range(W // 16):
                sl2 = pl.ds(cc * 16, 16)
                rows[buf, i, sl2] = rows[buf, i, sl2] * w_s
            return 0
        lax.fori_loop(0, B, _mul, 0)

    def head_body(h, _):
        def _z(i, _):
            for cc in range(W // 16):
                rows[0, i, pl.ds(cc * 16, 16)] = jnp.zeros((16,), jnp.float32)
                rows[1, i, pl.ds(cc * 16, 16)] = jnp.zeros((16,), jnp.float32)
            return 0
        lax.fori_loop(0, B, _z, 0)

        def _zd(i, _):
            den_v[pl.ds(i * 16, 16)] = jnp.zeros((16,), jnp.float32)
            return 0
        lax.fori_loop(0, NPAD // 16, _zd, 0)

        for j in range(ROWS_PER_TILE // (2 * B)):
            for u in range(2):
                sl = pl.ds(s * ROWS_PER_TILE + (2 * j + u) * B, B)
                pltpu.async_copy(rows.at[u], acc.at[sl], gsems[u])
        for j in range(ROWS_PER_TILE // (2 * B)):
            for u in range(2):
                sl = pl.ds(s * ROWS_PER_TILE + (2 * j + u) * B, B)
                pltpu.make_async_copy(rows.at[u], acc.at[sl], gsems[u]).wait()

        pltpu.sync_copy(elt.at[h], el_v)
        pltpu.sync_copy(ert.at[h], er_v)
        pltpu.sync_copy(emxt.at[h], emx_v)
        hoff = h * NPAD

        plsc.subcore_barrier()

        stage_start(0, 0)
        stage_wait(0, 0)
        gidx_fill(0, 0, 0, hoff)
        gather_start(0)

        def chunk_pair(i, _):
            for ch2 in range(2):
                ch = i * 2 + ch2
                cbuf = ch2
                ncbuf = 1 - ch2

                @pl.when(ch + 1 < NCH)
                def _():
                    stage_start(ch + 1, ncbuf)

                for bi in range(CHB):
                    b = ch * CHB + bi
                    j = bi % 2
                    nj = 1 - j

                    @pl.when(b + 1 < NB)
                    def _():
                        @pl.when(b >= 1)
                        def _():
                            scatter_wait(nj)
                        if bi == CHB - 1:
                            stage_wait(ch + 1, ncbuf)
                            gidx_fill(ncbuf, 0, nj, hoff)
                        else:
                            gidx_fill(cbuf, bi + 1, nj, hoff)
                        gather_start(nj)

                    gather_wait(j)
                    compute(cbuf, bi, j, hoff)
                    scatter_start(j)
            return 0
        lax.fori_loop(0, NCH // 2, chunk_pair, 0)

        scatter_wait(0)
        scatter_wait(1)
        plsc.subcore_barrier()

        row0 = (c * heads + h) * NPAD + s * ROWS_PER_TILE
        pltpu.sync_copy(acc.at[pl.ds(s * ROWS_PER_TILE, ROWS_PER_TILE)],
                        out2.at[pl.ds(row0, ROWS_PER_TILE)])
        pltpu.sync_copy(den_v, outd.at[pl.ds((wid * heads + h) * NPAD, NPAD)])
        plsc.subcore_barrier()
        return 0

    lax.fori_loop(0, heads, head_body, 0)


def _edge_call(heads):
    mesh = plsc.VectorSubcoreMesh(core_axis_name="c", subcore_axis_name="s")
    return pl.kernel(
        functools.partial(_edge_body, heads),
        out_type=(jax.ShapeDtypeStruct((2 * heads * NPAD, W), jnp.float32),
                  jax.ShapeDtypeStruct((NW * heads * NPAD,), jnp.float32)),
        mesh=mesh,
        compiler_params=pltpu.CompilerParams(needs_layout_passes=False),
        scratch_types=[
            pltpu.VMEM_SHARED((NPAD, W), jnp.float32),
            pltpu.VMEM((NPAD,), jnp.float32),
            pltpu.VMEM((NPAD,), jnp.float32),
            pltpu.VMEM((16,), jnp.float32),
            pltpu.VMEM((NPAD,), jnp.float32),
            pltpu.VMEM((2, CHB * B), jnp.int32),
            pltpu.VMEM((2, CHB * B), jnp.int32),
            pltpu.VMEM((2, B), jnp.int32),
            pltpu.VMEM((2, B, W), jnp.float32),
            pltpu.VMEM((2, B), jnp.int32),
            pltpu.VMEM((2, B, 16), jnp.float32),
            pltpu.SemaphoreType.DMA,
            pltpu.SemaphoreType.DMA,
            pltpu.SemaphoreType.DMA,
            pltpu.SemaphoreType.DMA,
            pltpu.SemaphoreType.DMA,
            pltpu.SemaphoreType.DMA,
        ],
    )


_EDGE_K = {8: _edge_call(8), 1: _edge_call(1)}


def _layer(x, srcp, dstp, Wm, al, ar, b, heads, act):
    feat3, el, er = _mm_proj(x, Wm, al, ar, heads)
    elmax = jnp.max(el, axis=0)
    emxt = jnp.broadcast_to(elmax[:, None], (heads, 16))
    pad = ((0, 0), (0, NPAD - N))
    elt = jnp.pad(el.T, pad)
    ert = jnp.pad(er.T, pad)
    out2, outd = _EDGE_K[heads](feat3.reshape(heads * NPAD, W), elt, ert, emxt,
                                srcp, dstp)
    o = out2.reshape(2, heads, NPAD, W).sum(axis=0)[:, :N, :]
    den = outd.reshape(NW, heads, NPAD)[:, :, :N].sum(axis=0)
    rst = o / (den[:, :, None] + 1e-9)
    rst = rst + b.reshape(heads, 1, 128)
    if act:
        rst = jax.nn.elu(rst)
    return rst


def kernel(inputs, edge_index, W0, al0, ar0, b0, W1, al1, ar1, b1, W2, al2, ar2, b2, W3, al3, ar3, b3):
    src = edge_index[0]
    dst = edge_index[1]
    srcp = jnp.concatenate([src, jnp.zeros((EPAD - E,), jnp.int32)])
    dstp = jnp.concatenate([dst, jnp.full((EPAD - E,), NPAD - 1, jnp.int32)])

    def nxt(rst):
        return rst.transpose(1, 0, 2).reshape(N, -1)

    h = nxt(_layer(inputs, srcp, dstp, W0, al0, ar0, b0, 8, True))
    h = nxt(_layer(h, srcp, dstp, W1, al1, ar1, b1, 8, True))
    h = nxt(_layer(h, srcp, dstp, W2, al2, ar2, b2, 8, True))
    h = _layer(h, srcp, dstp, W3, al3, ar3, b3, 1, False)[0]
    return h
